# baseline (speedup 1.0000x reference)
import functools

import jax
import jax.numpy as jnp
from jax import lax
from jax.experimental import pallas as pl
from jax.experimental.pallas import tpu as pltpu
from jax.experimental.pallas import tpu_sc as plsc

_R = 5
_NU = 10000
_NV = 1000
_D = 128
_H0 = 64
_H1 = 32
_BU = 4096
_BV = 1000
_KP = 1024

_NC = 2
_NS = 16
_NW = _NC * _NS
_GR_W = _BU // _NW
_CHUNK = 32
_NCHUNK = _GR_W // _CHUNK
_BVP = 1024
_VE_W = _BVP // _NW

_BUC = 400
_BLK = 256
_NB = _BU // _BLK


_NBLK = _NU // 8
_SBLK = (_NBLK + _NW - 1) // (2 * _NW)


def _sc_compress_body(ratings, code_out, in_a, in_b, out_a, out_b,
                      sem_ia, sem_ib, sem_oa, sem_ob):
    wid = lax.axis_index("s") * _NC + lax.axis_index("c")

    def start_in(blk, buf, sem):
        for r in range(_R):
            pltpu.async_copy(ratings.at[r, pl.ds(blk * 8, 8)], buf.at[r],
                             sem)

    def wait_in(blk, buf, sem):
        for r in range(_R):
            pltpu.make_async_copy(ratings.at[r, pl.ds(blk * 8, 8)],
                                  buf.at[r], sem).wait()

    def wait_out(buf, sem):
        pltpu.make_async_copy(buf, code_out.at[pl.ds(0, 8)], sem).wait()

    def compute8(buf, obuf):
        def chunk_rows(c, row_sub):
            for row in range(8):
                sl = pl.ds(c * 16 + row_sub * 16, 16)
                acc = buf[0, row, sl]
                for r in range(1, _R):
                    acc = acc + float(r + 1) * buf[r, row, sl]
                obuf[row, pl.ds(c * 16 + row_sub * 16, 16)] = acc

        def pair(k, carry):
            chunk_rows(2 * k, 0)
            chunk_rows(2 * k, 1)
            return carry

        lax.fori_loop(0, _NV // 32, pair, 0)
        z16 = jnp.zeros((16,), jnp.float32)
        for row in range(8):
            sl = pl.ds(_NV - 16, 16)
            acc = buf[0, row, sl]
            for r in range(1, _R):
                acc = acc + float(r + 1) * buf[r, row, sl]
            obuf[row, pl.ds(_NV - 16, 16)] = acc
            obuf[row, pl.ds(_NV, 16)] = z16
            obuf[row, pl.ds(_KP - 16, 16)] = z16

    start_in(wid, in_a, sem_ia)

    def body(s, carry):
        blk_a = wid + 2 * _NW * s
        blk_b = blk_a + _NW
        blk_a2 = blk_a + 2 * _NW
        valid_b = blk_b < _NBLK

        @pl.when(valid_b)
        def _():
            start_in(blk_b, in_b, sem_ib)

        wait_in(blk_a, in_a, sem_ia)

        @pl.when(s > 0)
        def _():
            wait_out(out_a, sem_oa)

        compute8(in_a, out_a)
        pltpu.async_copy(out_a, code_out.at[pl.ds(blk_a * 8, 8)], sem_oa)

        @pl.when(blk_a2 < _NBLK)
        def _():
            start_in(blk_a2, in_a, sem_ia)

        @pl.when(valid_b)
        def _():
            wait_in(blk_b, in_b, sem_ib)

            @pl.when(s > 0)
            def _():
                wait_out(out_b, sem_ob)

            compute8(in_b, out_b)
            pltpu.async_copy(out_b, code_out.at[pl.ds(blk_b * 8, 8)],
                             sem_ob)

        return carry

    lax.fori_loop(0, _SBLK, body, 0)

    wait_out(out_a, sem_oa)
    wait_out(out_b, sem_ob)


_sc_compress = functools.partial(
    pl.kernel,
    mesh=plsc.VectorSubcoreMesh(core_axis_name="c", subcore_axis_name="s"),
    out_type=jax.ShapeDtypeStruct((_NU, _KP), jnp.float32),
    scratch_types=[
        pltpu.VMEM((_R, 8, _NV), jnp.float32),
        pltpu.VMEM((_R, 8, _NV), jnp.float32),
        pltpu.VMEM((8, _KP), jnp.float32),
        pltpu.VMEM((8, _KP), jnp.float32),
        pltpu.SemaphoreType.DMA,
        pltpu.SemaphoreType.DMA,
        pltpu.SemaphoreType.DMA,
        pltpu.SemaphoreType.DMA,
    ],
)(_sc_compress_body)


def _sc_gather_body(code, u2, v2, u_table, v_table,
                    gc_out, ue_out, ve_out,
                    uidx, vidx, rows_a, rows_b, erows, vrows,
                    sem_a, sem_b, sem_e):
    wid = lax.axis_index("s") * _NC + lax.axis_index("c")
    base = wid * _GR_W

    pltpu.sync_copy(u2.at[wid], uidx)

    pltpu.async_copy(u_table.at[uidx], erows, sem_e).wait()
    pltpu.sync_copy(erows, ue_out.at[pl.ds(base, _GR_W)])

    pltpu.sync_copy(v2.at[wid], vidx)
    pltpu.async_copy(v_table.at[vidx], vrows, sem_e).wait()
    pltpu.sync_copy(vrows, ve_out.at[pl.ds(wid * _VE_W, _VE_W)])

    bufs = (rows_a, rows_b)
    sems = (sem_a, sem_b)
    handles = [None, None]
    handles[0] = pltpu.async_copy(
        code.at[uidx.at[pl.ds(0, _CHUNK)]], rows_a, sem_a)
    for c in range(_NCHUNK):
        if c + 1 < _NCHUNK:
            handles[(c + 1) % 2] = pltpu.async_copy(
                code.at[uidx.at[pl.ds((c + 1) * _CHUNK, _CHUNK)]],
                bufs[(c + 1) % 2], sems[(c + 1) % 2])
        handles[c % 2].wait()
        pltpu.sync_copy(bufs[c % 2],
                        gc_out.at[pl.ds(base + c * _CHUNK, _CHUNK)])


_sc_gather = functools.partial(
    pl.kernel,
    mesh=plsc.VectorSubcoreMesh(core_axis_name="c", subcore_axis_name="s"),
    out_type=[
        jax.ShapeDtypeStruct((_BU, _KP), jnp.float32),
        jax.ShapeDtypeStruct((_BU, _D), jnp.float32),
        jax.ShapeDtypeStruct((_BVP, _D), jnp.float32),
    ],
    scratch_types=[
        pltpu.VMEM((_GR_W,), jnp.int32),
        pltpu.VMEM((_VE_W,), jnp.int32),
        pltpu.VMEM((_CHUNK, _KP), jnp.float32),
        pltpu.VMEM((_CHUNK, _KP), jnp.float32),
        pltpu.VMEM((_GR_W, _D), jnp.float32),
        pltpu.VMEM((_VE_W, _D), jnp.float32),
        pltpu.SemaphoreType.DMA,
        pltpu.SemaphoreType.DMA,
        pltpu.SemaphoreType.DMA,
    ],
)(_sc_gather_body)


def _main_body(gc_ref, ue_ref, vemb_ref, gclW_ref, v_ref, dW_ref, db_ref,
               gclb_ref, bw_ref,
               mhat_ref, loss_ref, acc_ref,
               bv_s, cnt_s, msgu_s, di_s, cs_s, tv_s, tcode_s, uh_s, vh_s,
               sacc):
    i = pl.program_id(0)

    @pl.when(i == 0)
    def _prep():
        kio = lax.broadcasted_iota(jnp.int32, (_KP, _BV), 0)
        G = (kio == v_ref[...]).astype(jnp.float32)
        cnt_s[...] = jnp.sum(G, axis=1, keepdims=True).astype(jnp.bfloat16)
        ve = vemb_ref[...]
        for r in range(_R):
            sv = jnp.dot(ve, gclW_ref[r], preferred_element_type=jnp.float32)
            bv_s[r] = jnp.dot(G, sv,
                              preferred_element_type=jnp.float32
                              ).astype(jnp.bfloat16)

    @pl.when(i < _NB)
    def _agg():
        ue = ue_ref[...]
        gc = gc_ref[...]
        ones_row = jnp.ones((1, _BLK), jnp.bfloat16)
        acc_msg = jnp.zeros((_BLK, _H0), jnp.float32)
        acc_di = jnp.zeros((_BLK, 1), jnp.float32)
        acc_cs = jnp.zeros((1, _KP), jnp.float32)
        acc_tv = jnp.zeros((_KP, _H0), jnp.float32)
        cnt_col = cnt_s[...]
        for r in range(_R):
            Rr = (gc == float(r + 1)).astype(jnp.bfloat16)
            su = jnp.dot(ue, gclW_ref[r], preferred_element_type=jnp.float32
                         ).astype(jnp.bfloat16)
            acc_msg = acc_msg + jnp.dot(Rr, bv_s[r],
                                        preferred_element_type=jnp.float32)
            acc_di = acc_di + jnp.dot(Rr, cnt_col,
                                      preferred_element_type=jnp.float32)
            acc_cs = acc_cs + lax.dot_general(
                ones_row, Rr, (((1,), (0,)), ((), ())),
                preferred_element_type=jnp.float32)
            acc_tv = acc_tv + lax.dot_general(
                Rr, su, (((0,), (0,)), ((), ())),
                preferred_element_type=jnp.float32)

        kio = lax.broadcasted_iota(jnp.int32, (_KP, _BV), 0)
        Gb = (kio == v_ref[...]).astype(jnp.bfloat16)
        tcode = lax.dot_general(
            gc.astype(jnp.bfloat16), Gb, (((1,), (0,)), ((), ())),
            preferred_element_type=jnp.float32)
        tcode_s[pl.ds(i * _BLK, _BLK)] = tcode.astype(jnp.bfloat16)
        msgu_s[pl.ds(i * _BLK, _BLK)] = acc_msg
        di_s[pl.ds(i * _BLK, _BLK)] = acc_di

        @pl.when(i == 0)
        def _():
            cs_s[...] = acc_cs
            tv_s[...] = acc_tv

        @pl.when(i != 0)
        def _():
            cs_s[...] = cs_s[...] + acc_cs
            tv_s[...] = tv_s[...] + acc_tv

    @pl.when(i >= _NB)
    def _decode():
        j = i - _NB

        @pl.when(j == 0)
        def _hidden():
            kio = lax.broadcasted_iota(jnp.int32, (_KP, _BV), 0)
            G = (kio == v_ref[...]).astype(jnp.float32)
            du = lax.dot_general(G, cs_s[...], (((0,), (1,)), ((), ())),
                                 preferred_element_type=jnp.float32)
            msgv = lax.dot_general(G, tv_s[...], (((0,), (0,)), ((), ())),
                                   preferred_element_type=jnp.float32)
            deg = jnp.concatenate([du, di_s[...]], axis=0)
            c = jnp.where(deg > 0, 1.0 / jnp.where(deg > 0, deg, 1.0), 0.0)
            cu = c[:_BU]
            ci = c[_BU:]
            bsum = jnp.sum(gclb_ref[...], axis=0, keepdims=True)
            zu = jnp.maximum(msgu_s[...] * cu + bsum, 0.0)
            zv = jnp.maximum(msgv * ci + bsum, 0.0)
            dW = dW_ref[...]
            db = db_ref[...]
            uh_s[...] = jax.nn.sigmoid(
                jnp.dot(zu, dW, preferred_element_type=jnp.float32) + db)
            vh_s[...] = jax.nn.sigmoid(
                jnp.dot(zv, dW, preferred_element_type=jnp.float32) + db)

        uh = uh_s[pl.ds(j * _BLK, _BLK)]
        vh = vh_s[...]
        Os = []
        for r in range(_R):
            A = jnp.dot(uh, bw_ref[r], preferred_element_type=jnp.float32)
            Os.append(lax.dot_general(A, vh, (((1,), (1,)), ((), ())),
                                      preferred_element_type=jnp.float32))
        mx = Os[0]
        for r in range(1, _R):
            mx = jnp.maximum(mx, Os[r])
        es = [jnp.exp(o - mx) for o in Os]
        se = es[0]
        for r in range(1, _R):
            se = se + es[r]
        num = jnp.zeros_like(se)
        for r in range(1, _R):
            num = num + float(r) * es[r]
        mhat_ref[...] = num / se

        tc = tcode_s[pl.ds(j * _BLK, _BLK)].astype(jnp.float32)
        obs = tc > 0.5
        ot = jnp.zeros_like(mx)
        for r in range(_R):
            ot = jnp.where(tc == float(r + 1), Os[r], ot)
        lterm = jnp.where(obs, mx + jnp.log(se) - ot, 0.0)

        pbest = Os[0]
        pcls = jnp.zeros_like(mx)
        for r in range(1, _R):
            gt = Os[r] > pbest
            pbest = jnp.where(gt, Os[r], pbest)
            pcls = jnp.where(gt, float(r), pcls)
        corr = jnp.where(obs & (pcls == (tc - 1.0)), 1.0, 0.0)

        ls = jnp.sum(lterm)
        nb = jnp.sum(jnp.where(obs, 1.0, 0.0))
        cr = jnp.sum(corr)

        @pl.when(j == 0)
        def _():
            sacc[0] = ls
            sacc[1] = nb
            sacc[2] = cr

        @pl.when(j != 0)
        def _():
            sacc[0] = sacc[0] + ls
            sacc[1] = sacc[1] + nb
            sacc[2] = sacc[2] + cr

        @pl.when(j == _NB - 1)
        def _():
            nbm = jnp.maximum(sacc[1], 1.0)
            loss_ref[...] = jnp.broadcast_to(sacc[0] / nbm, (1, 1))
            acc_ref[...] = jnp.broadcast_to(sacc[2] / nbm, (1, 1))


def _main_call(gcode, uemb, vemb, gcl_W, v_row, dense_W, db_row, gcl_b,
               bilin_W):
    return pl.pallas_call(
        _main_body,
        grid=(2 * _NB,),
        in_specs=[
            pl.BlockSpec((_BLK, _KP), lambda i: (jnp.minimum(i, _NB - 1), 0)),
            pl.BlockSpec((_BLK, _D), lambda i: (jnp.minimum(i, _NB - 1), 0)),
            pl.BlockSpec((_NV, _D), lambda i: (0, 0)),
            pl.BlockSpec((_R, _D, _H0), lambda i: (0, 0, 0)),
            pl.BlockSpec((1, _BV), lambda i: (0, 0)),
            pl.BlockSpec((_H0, _H1), lambda i: (0, 0)),
            pl.BlockSpec((1, _H1), lambda i: (0, 0)),
            pl.BlockSpec((_R, _H0), lambda i: (0, 0)),
            pl.BlockSpec((_R, _H1, _H1), lambda i: (0, 0, 0)),
        ],
        out_specs=[
            pl.BlockSpec((_BLK, _BV), lambda i: (jnp.maximum(i - _NB, 0), 0)),
            pl.BlockSpec((1, 1), lambda i: (0, 0)),
            pl.BlockSpec((1, 1), lambda i: (0, 0)),
        ],
        out_shape=[
            jax.ShapeDtypeStruct((_BU, _BV), jnp.float32),
            jax.ShapeDtypeStruct((1, 1), jnp.float32),
            jax.ShapeDtypeStruct((1, 1), jnp.float32),
        ],
        scratch_shapes=[
            pltpu.VMEM((_R, _KP, _H0), jnp.bfloat16),
            pltpu.VMEM((_KP, 1), jnp.bfloat16),
            pltpu.VMEM((_BU, _H0), jnp.float32),
            pltpu.VMEM((_BU, 1), jnp.float32),
            pltpu.VMEM((1, _KP), jnp.float32),
            pltpu.VMEM((_KP, _H0), jnp.float32),
            pltpu.VMEM((_BU, _BV), jnp.bfloat16),
            pltpu.VMEM((_BU, _H1), jnp.float32),
            pltpu.VMEM((_NV, _H1), jnp.float32),
            pltpu.SMEM((3,), jnp.float32),
        ],
        compiler_params=pltpu.CompilerParams(
            dimension_semantics=("arbitrary",)),
    )(gcode, uemb, vemb, gcl_W, v_row, dense_W, db_row, gcl_b, bilin_W)


def kernel(u, v, u_table, v_table, gcl_W, gcl_b, dense_W, dense_b, bilin_W,
           ratings):
    u = u.astype(jnp.int32)
    v = v.astype(jnp.int32)
    u2 = u.reshape(_NW, _GR_W)
    v2 = jnp.concatenate([v, jnp.zeros((_BVP - _BV,), jnp.int32)]
                         ).reshape(_NW, _VE_W)

    code = _sc_compress(ratings)
    gcode, uemb, vemb_p = _sc_gather(code, u2, v2, u_table, v_table)
    vemb = vemb_p[:_NV]
    v_row = v.reshape(1, _BV)

    mhat, loss, acc = _main_call(gcode, uemb, vemb, gcl_W, v_row, dense_W,
                                 dense_b.reshape(1, _H1), gcl_b, bilin_W)
    return mhat, loss[0, 0], acc[0, 0]


# speedup vs baseline: 1.3073x; 1.3073x over previous
import functools

import jax
import jax.numpy as jnp
from jax import lax
from jax.experimental import pallas as pl
from jax.experimental.pallas import tpu as pltpu
from jax.experimental.pallas import tpu_sc as plsc

_R = 5
_NU = 10000
_NV = 1000
_D = 128
_H0 = 64
_H1 = 32
_BU = 4096
_BV = 1000
_KP = 1024

_NC = 2
_NS = 16
_NW = _NC * _NS
_GR_W = _BU // _NW
_CHUNK = 32
_NCHUNK = _GR_W // _CHUNK
_BVP = 1024
_VE_W = _BVP // _NW

_BUC = 400
_BLK = 256
_NB = _BU // _BLK


def _compress_body(ratings_ref, code_ref):
    acc = ratings_ref[0]
    for r in range(1, _R):
        acc = acc + float(r + 1) * ratings_ref[r]
    code_ref[...] = jnp.zeros((_BUC, _KP), jnp.float32)
    code_ref[:, : _NV] = acc


def _compress_call(ratings):
    n = _NU // _BUC
    return pl.pallas_call(
        _compress_body,
        grid=(n,),
        in_specs=[pl.BlockSpec((_R, _BUC, _NV), lambda i: (0, i, 0))],
        out_specs=pl.BlockSpec((_BUC, _KP), lambda i: (i, 0)),
        out_shape=jax.ShapeDtypeStruct((_NU, _KP), jnp.float32),
        compiler_params=pltpu.CompilerParams(
            dimension_semantics=("arbitrary",)),
    )(ratings)


def _sc_gather_body(code, u2, v2, u_table, v_table,
                    gc_out, ue_out, ve_out,
                    uidx, vidx, rows_a, rows_b, erows, vrows,
                    sem_a, sem_b, sem_e):
    wid = lax.axis_index("s") * _NC + lax.axis_index("c")
    base = wid * _GR_W

    pltpu.sync_copy(u2.at[wid], uidx)

    pltpu.async_copy(u_table.at[uidx], erows, sem_e).wait()
    pltpu.sync_copy(erows, ue_out.at[pl.ds(base, _GR_W)])

    pltpu.sync_copy(v2.at[wid], vidx)
    pltpu.async_copy(v_table.at[vidx], vrows, sem_e).wait()
    pltpu.sync_copy(vrows, ve_out.at[pl.ds(wid * _VE_W, _VE_W)])

    bufs = (rows_a, rows_b)
    sems = (sem_a, sem_b)
    handles = [None, None]
    handles[0] = pltpu.async_copy(
        code.at[uidx.at[pl.ds(0, _CHUNK)]], rows_a, sem_a)
    for c in range(_NCHUNK):
        if c + 1 < _NCHUNK:
            handles[(c + 1) % 2] = pltpu.async_copy(
                code.at[uidx.at[pl.ds((c + 1) * _CHUNK, _CHUNK)]],
                bufs[(c + 1) % 2], sems[(c + 1) % 2])
        handles[c % 2].wait()
        pltpu.sync_copy(bufs[c % 2],
                        gc_out.at[pl.ds(base + c * _CHUNK, _CHUNK)])


_sc_gather = functools.partial(
    pl.kernel,
    mesh=plsc.VectorSubcoreMesh(core_axis_name="c", subcore_axis_name="s"),
    out_type=[
        jax.ShapeDtypeStruct((_BU, _KP), jnp.float32),
        jax.ShapeDtypeStruct((_BU, _D), jnp.float32),
        jax.ShapeDtypeStruct((_BVP, _D), jnp.float32),
    ],
    scratch_types=[
        pltpu.VMEM((_GR_W,), jnp.int32),
        pltpu.VMEM((_VE_W,), jnp.int32),
        pltpu.VMEM((_CHUNK, _KP), jnp.float32),
        pltpu.VMEM((_CHUNK, _KP), jnp.float32),
        pltpu.VMEM((_GR_W, _D), jnp.float32),
        pltpu.VMEM((_VE_W, _D), jnp.float32),
        pltpu.SemaphoreType.DMA,
        pltpu.SemaphoreType.DMA,
        pltpu.SemaphoreType.DMA,
    ],
)(_sc_gather_body)


def _main_body(gc_ref, ue_ref, vemb_ref, gclW_ref, v_ref, dW_ref, db_ref,
               gclb_ref, bw_ref,
               mhat_ref, loss_ref, acc_ref,
               bv_s, cnt_s, msgu_s, di_s, cs_s, tv_s, tcode_s, uh_s, vh_s,
               sacc):
    i = pl.program_id(0)

    @pl.when(i == 0)
    def _prep():
        kio = lax.broadcasted_iota(jnp.int32, (_KP, _BV), 0)
        G = (kio == v_ref[...]).astype(jnp.float32)
        cnt_s[...] = jnp.sum(G, axis=1, keepdims=True)
        ve = vemb_ref[...]
        for r in range(_R):
            sv = jnp.dot(ve, gclW_ref[r], preferred_element_type=jnp.float32)
            bv_s[r] = jnp.dot(G, sv, preferred_element_type=jnp.float32)

    @pl.when(i < _NB)
    def _agg():
        ue = ue_ref[...]
        gc = gc_ref[...]
        acc_msg = jnp.zeros((_BLK, _H0), jnp.float32)
        acc_di = jnp.zeros((_BLK, 1), jnp.float32)
        acc_cs = jnp.zeros((1, _KP), jnp.float32)
        acc_tv = jnp.zeros((_KP, _H0), jnp.float32)
        cnt_col = cnt_s[...]
        for r in range(_R):
            Rr = (gc == float(r + 1)).astype(jnp.float32)
            su = jnp.dot(ue, gclW_ref[r], preferred_element_type=jnp.float32)
            acc_msg = acc_msg + jnp.dot(Rr, bv_s[r],
                                        preferred_element_type=jnp.float32)
            acc_di = acc_di + jnp.dot(Rr, cnt_col,
                                      preferred_element_type=jnp.float32)
            acc_cs = acc_cs + jnp.sum(Rr, axis=0, keepdims=True)
            acc_tv = acc_tv + lax.dot_general(
                Rr, su, (((0,), (0,)), ((), ())),
                preferred_element_type=jnp.float32)

        kio = lax.broadcasted_iota(jnp.int32, (_KP, _BV), 0)
        Gb = (kio == v_ref[...]).astype(jnp.bfloat16)
        tcode = lax.dot_general(
            gc.astype(jnp.bfloat16), Gb, (((1,), (0,)), ((), ())),
            preferred_element_type=jnp.float32)
        tcode_s[pl.ds(i * _BLK, _BLK)] = tcode.astype(jnp.bfloat16)
        msgu_s[pl.ds(i * _BLK, _BLK)] = acc_msg
        di_s[pl.ds(i * _BLK, _BLK)] = acc_di

        @pl.when(i == 0)
        def _():
            cs_s[...] = acc_cs
            tv_s[...] = acc_tv

        @pl.when(i != 0)
        def _():
            cs_s[...] = cs_s[...] + acc_cs
            tv_s[...] = tv_s[...] + acc_tv

    @pl.when(i >= _NB)
    def _decode():
        j = i - _NB

        @pl.when(j == 0)
        def _hidden():
            kio = lax.broadcasted_iota(jnp.int32, (_KP, _BV), 0)
            G = (kio == v_ref[...]).astype(jnp.float32)
            du = lax.dot_general(G, cs_s[...], (((0,), (1,)), ((), ())),
                                 preferred_element_type=jnp.float32)
            msgv = lax.dot_general(G, tv_s[...], (((0,), (0,)), ((), ())),
                                   preferred_element_type=jnp.float32)
            deg = jnp.concatenate([du, di_s[...]], axis=0)
            c = jnp.where(deg > 0, 1.0 / jnp.where(deg > 0, deg, 1.0), 0.0)
            cu = c[:_BU]
            ci = c[_BU:]
            bsum = jnp.sum(gclb_ref[...], axis=0, keepdims=True)
            zu = jnp.maximum(msgu_s[...] * cu + bsum, 0.0)
            zv = jnp.maximum(msgv * ci + bsum, 0.0)
            dW = dW_ref[...]
            db = db_ref[...]
            uh_s[...] = jax.nn.sigmoid(
                jnp.dot(zu, dW, preferred_element_type=jnp.float32) + db)
            vh_s[...] = jax.nn.sigmoid(
                jnp.dot(zv, dW, preferred_element_type=jnp.float32) + db)

        uh = uh_s[pl.ds(j * _BLK, _BLK)]
        vh = vh_s[...]
        Os = []
        for r in range(_R):
            A = jnp.dot(uh, bw_ref[r], preferred_element_type=jnp.float32)
            Os.append(lax.dot_general(A, vh, (((1,), (1,)), ((), ())),
                                      preferred_element_type=jnp.float32))
        mx = Os[0]
        for r in range(1, _R):
            mx = jnp.maximum(mx, Os[r])
        es = [jnp.exp(o - mx) for o in Os]
        se = es[0]
        for r in range(1, _R):
            se = se + es[r]
        num = jnp.zeros_like(se)
        for r in range(1, _R):
            num = num + float(r) * es[r]
        mhat_ref[...] = num / se

        tc = tcode_s[pl.ds(j * _BLK, _BLK)].astype(jnp.float32)
        obs = tc > 0.5
        ot = jnp.zeros_like(mx)
        for r in range(_R):
            ot = jnp.where(tc == float(r + 1), Os[r], ot)
        lterm = jnp.where(obs, mx + jnp.log(se) - ot, 0.0)

        pbest = Os[0]
        pcls = jnp.zeros_like(mx)
        for r in range(1, _R):
            gt = Os[r] > pbest
            pbest = jnp.where(gt, Os[r], pbest)
            pcls = jnp.where(gt, float(r), pcls)
        corr = jnp.where(obs & (pcls == (tc - 1.0)), 1.0, 0.0)

        ls = jnp.sum(lterm)
        nb = jnp.sum(jnp.where(obs, 1.0, 0.0))
        cr = jnp.sum(corr)

        @pl.when(j == 0)
        def _():
            sacc[0] = ls
            sacc[1] = nb
            sacc[2] = cr

        @pl.when(j != 0)
        def _():
            sacc[0] = sacc[0] + ls
            sacc[1] = sacc[1] + nb
            sacc[2] = sacc[2] + cr

        @pl.when(j == _NB - 1)
        def _():
            nbm = jnp.maximum(sacc[1], 1.0)
            loss_ref[...] = jnp.broadcast_to(sacc[0] / nbm, (1, 1))
            acc_ref[...] = jnp.broadcast_to(sacc[2] / nbm, (1, 1))


def _main_call(gcode, uemb, vemb, gcl_W, v_row, dense_W, db_row, gcl_b,
               bilin_W):
    return pl.pallas_call(
        _main_body,
        grid=(2 * _NB,),
        in_specs=[
            pl.BlockSpec((_BLK, _KP), lambda i: (jnp.minimum(i, _NB - 1), 0)),
            pl.BlockSpec((_BLK, _D), lambda i: (jnp.minimum(i, _NB - 1), 0)),
            pl.BlockSpec((_NV, _D), lambda i: (0, 0)),
            pl.BlockSpec((_R, _D, _H0), lambda i: (0, 0, 0)),
            pl.BlockSpec((1, _BV), lambda i: (0, 0)),
            pl.BlockSpec((_H0, _H1), lambda i: (0, 0)),
            pl.BlockSpec((1, _H1), lambda i: (0, 0)),
            pl.BlockSpec((_R, _H0), lambda i: (0, 0)),
            pl.BlockSpec((_R, _H1, _H1), lambda i: (0, 0, 0)),
        ],
        out_specs=[
            pl.BlockSpec((_BLK, _BV), lambda i: (jnp.maximum(i - _NB, 0), 0)),
            pl.BlockSpec((1, 1), lambda i: (0, 0)),
            pl.BlockSpec((1, 1), lambda i: (0, 0)),
        ],
        out_shape=[
            jax.ShapeDtypeStruct((_BU, _BV), jnp.float32),
            jax.ShapeDtypeStruct((1, 1), jnp.float32),
            jax.ShapeDtypeStruct((1, 1), jnp.float32),
        ],
        scratch_shapes=[
            pltpu.VMEM((_R, _KP, _H0), jnp.float32),
            pltpu.VMEM((_KP, 1), jnp.float32),
            pltpu.VMEM((_BU, _H0), jnp.float32),
            pltpu.VMEM((_BU, 1), jnp.float32),
            pltpu.VMEM((1, _KP), jnp.float32),
            pltpu.VMEM((_KP, _H0), jnp.float32),
            pltpu.VMEM((_BU, _BV), jnp.bfloat16),
            pltpu.VMEM((_BU, _H1), jnp.float32),
            pltpu.VMEM((_NV, _H1), jnp.float32),
            pltpu.SMEM((3,), jnp.float32),
        ],
        compiler_params=pltpu.CompilerParams(
            dimension_semantics=("arbitrary",)),
    )(gcode, uemb, vemb, gcl_W, v_row, dense_W, db_row, gcl_b, bilin_W)


def kernel(u, v, u_table, v_table, gcl_W, gcl_b, dense_W, dense_b, bilin_W,
           ratings):
    u = u.astype(jnp.int32)
    v = v.astype(jnp.int32)
    u2 = u.reshape(_NW, _GR_W)
    v2 = jnp.concatenate([v, jnp.zeros((_BVP - _BV,), jnp.int32)]
                         ).reshape(_NW, _VE_W)

    code = _compress_call(ratings)
    gcode, uemb, vemb_p = _sc_gather(code, u2, v2, u_table, v_table)
    vemb = vemb_p[:_NV]
    v_row = v.reshape(1, _BV)

    mhat, loss, acc = _main_call(gcode, uemb, vemb, gcl_W, v_row, dense_W,
                                 dense_b.reshape(1, _H1), gcl_b, bilin_W)
    return mhat, loss[0, 0], acc[0, 0]


# speedup vs baseline: 1.3073x; 1.0001x over previous
import functools

import jax
import jax.numpy as jnp
from jax import lax
from jax.experimental import pallas as pl
from jax.experimental.pallas import tpu as pltpu
from jax.experimental.pallas import tpu_sc as plsc

_R = 5
_NU = 10000
_NV = 1000
_D = 128
_H0 = 64
_H1 = 32
_BU = 4096
_BV = 1000
_KP = 1024

_NC = 2
_NS = 16
_NW = _NC * _NS
_GR_W = _BU // _NW
_CHUNK = 32
_NCHUNK = _GR_W // _CHUNK
_BVP = 1024
_VE_W = _BVP // _NW

_BUC = 400
_BLK = 256
_NB = _BU // _BLK


def _compress_body(ratings_ref, code_ref):
    acc = ratings_ref[0]
    for r in range(1, _R):
        acc = acc + float(r + 1) * ratings_ref[r]
    code_ref[...] = jnp.zeros((_BUC, _KP), jnp.float32)
    code_ref[:, : _NV] = acc


def _compress_call(ratings):
    n = _NU // _BUC
    return pl.pallas_call(
        _compress_body,
        grid=(n,),
        in_specs=[pl.BlockSpec((_R, _BUC, _NV), lambda i: (0, i, 0))],
        out_specs=pl.BlockSpec((_BUC, _KP), lambda i: (i, 0)),
        out_shape=jax.ShapeDtypeStruct((_NU, _KP), jnp.float32),
        compiler_params=pltpu.CompilerParams(
            dimension_semantics=("parallel",)),
    )(ratings)


def _sc_gather_body(code, u2, v2, u_table, v_table,
                    gc_out, ue_out, ve_out,
                    uidx, vidx, rows_a, rows_b, erows, vrows,
                    sem_a, sem_b, sem_e):
    wid = lax.axis_index("s") * _NC + lax.axis_index("c")
    base = wid * _GR_W

    pltpu.sync_copy(u2.at[wid], uidx)

    pltpu.async_copy(u_table.at[uidx], erows, sem_e).wait()
    pltpu.sync_copy(erows, ue_out.at[pl.ds(base, _GR_W)])

    pltpu.sync_copy(v2.at[wid], vidx)
    pltpu.async_copy(v_table.at[vidx], vrows, sem_e).wait()
    pltpu.sync_copy(vrows, ve_out.at[pl.ds(wid * _VE_W, _VE_W)])

    bufs = (rows_a, rows_b)
    sems = (sem_a, sem_b)
    handles = [None, None]
    handles[0] = pltpu.async_copy(
        code.at[uidx.at[pl.ds(0, _CHUNK)]], rows_a, sem_a)
    for c in range(_NCHUNK):
        if c + 1 < _NCHUNK:
            handles[(c + 1) % 2] = pltpu.async_copy(
                code.at[uidx.at[pl.ds((c + 1) * _CHUNK, _CHUNK)]],
                bufs[(c + 1) % 2], sems[(c + 1) % 2])
        handles[c % 2].wait()
        pltpu.sync_copy(bufs[c % 2],
                        gc_out.at[pl.ds(base + c * _CHUNK, _CHUNK)])


_sc_gather = functools.partial(
    pl.kernel,
    mesh=plsc.VectorSubcoreMesh(core_axis_name="c", subcore_axis_name="s"),
    out_type=[
        jax.ShapeDtypeStruct((_BU, _KP), jnp.float32),
        jax.ShapeDtypeStruct((_BU, _D), jnp.float32),
        jax.ShapeDtypeStruct((_BVP, _D), jnp.float32),
    ],
    scratch_types=[
        pltpu.VMEM((_GR_W,), jnp.int32),
        pltpu.VMEM((_VE_W,), jnp.int32),
        pltpu.VMEM((_CHUNK, _KP), jnp.float32),
        pltpu.VMEM((_CHUNK, _KP), jnp.float32),
        pltpu.VMEM((_GR_W, _D), jnp.float32),
        pltpu.VMEM((_VE_W, _D), jnp.float32),
        pltpu.SemaphoreType.DMA,
        pltpu.SemaphoreType.DMA,
        pltpu.SemaphoreType.DMA,
    ],
)(_sc_gather_body)


def _main_body(gc_ref, ue_ref, vemb_ref, gclW_ref, v_ref, dW_ref, db_ref,
               gclb_ref, bw_ref,
               mhat_ref, loss_ref, acc_ref,
               bv_s, cnt_s, msgu_s, di_s, cs_s, tv_s, tcode_s, uh_s, vh_s,
               sacc):
    i = pl.program_id(0)

    @pl.when(i == 0)
    def _prep():
        kio = lax.broadcasted_iota(jnp.int32, (_KP, _BV), 0)
        G = (kio == v_ref[...]).astype(jnp.float32)
        cnt_s[...] = jnp.sum(G, axis=1, keepdims=True)
        ve = vemb_ref[...]
        for r in range(_R):
            sv = jnp.dot(ve, gclW_ref[r], preferred_element_type=jnp.float32)
            bv_s[r] = jnp.dot(G, sv, preferred_element_type=jnp.float32)

    @pl.when(i < _NB)
    def _agg():
        ue = ue_ref[...]
        gc = gc_ref[...]
        acc_msg = jnp.zeros((_BLK, _H0), jnp.float32)
        acc_di = jnp.zeros((_BLK, 1), jnp.float32)
        acc_cs = jnp.zeros((1, _KP), jnp.float32)
        acc_tv = jnp.zeros((_KP, _H0), jnp.float32)
        cnt_col = cnt_s[...]
        for r in range(_R):
            Rr = (gc == float(r + 1)).astype(jnp.float32)
            su = jnp.dot(ue, gclW_ref[r], preferred_element_type=jnp.float32)
            acc_msg = acc_msg + jnp.dot(Rr, bv_s[r],
                                        preferred_element_type=jnp.float32)
            acc_di = acc_di + jnp.dot(Rr, cnt_col,
                                      preferred_element_type=jnp.float32)
            acc_cs = acc_cs + jnp.sum(Rr, axis=0, keepdims=True)
            acc_tv = acc_tv + lax.dot_general(
                Rr, su, (((0,), (0,)), ((), ())),
                preferred_element_type=jnp.float32)

        kio = lax.broadcasted_iota(jnp.int32, (_KP, _BV), 0)
        Gb = (kio == v_ref[...]).astype(jnp.bfloat16)
        tcode = lax.dot_general(
            gc.astype(jnp.bfloat16), Gb, (((1,), (0,)), ((), ())),
            preferred_element_type=jnp.float32)
        tcode_s[pl.ds(i * _BLK, _BLK)] = tcode.astype(jnp.bfloat16)
        msgu_s[pl.ds(i * _BLK, _BLK)] = acc_msg
        di_s[pl.ds(i * _BLK, _BLK)] = acc_di

        @pl.when(i == 0)
        def _():
            cs_s[...] = acc_cs
            tv_s[...] = acc_tv

        @pl.when(i != 0)
        def _():
            cs_s[...] = cs_s[...] + acc_cs
            tv_s[...] = tv_s[...] + acc_tv

    @pl.when(i >= _NB)
    def _decode():
        j = i - _NB

        @pl.when(j == 0)
        def _hidden():
            kio = lax.broadcasted_iota(jnp.int32, (_KP, _BV), 0)
            G = (kio == v_ref[...]).astype(jnp.float32)
            du = lax.dot_general(G, cs_s[...], (((0,), (1,)), ((), ())),
                                 preferred_element_type=jnp.float32)
            msgv = lax.dot_general(G, tv_s[...], (((0,), (0,)), ((), ())),
                                   preferred_element_type=jnp.float32)
            deg = jnp.concatenate([du, di_s[...]], axis=0)
            c = jnp.where(deg > 0, 1.0 / jnp.where(deg > 0, deg, 1.0), 0.0)
            cu = c[:_BU]
            ci = c[_BU:]
            bsum = jnp.sum(gclb_ref[...], axis=0, keepdims=True)
            zu = jnp.maximum(msgu_s[...] * cu + bsum, 0.0)
            zv = jnp.maximum(msgv * ci + bsum, 0.0)
            dW = dW_ref[...]
            db = db_ref[...]
            uh_s[...] = jax.nn.sigmoid(
                jnp.dot(zu, dW, preferred_element_type=jnp.float32) + db)
            vh_s[...] = jax.nn.sigmoid(
                jnp.dot(zv, dW, preferred_element_type=jnp.float32) + db)

        uh = uh_s[pl.ds(j * _BLK, _BLK)]
        vh = vh_s[...]
        Os = []
        for r in range(_R):
            A = jnp.dot(uh, bw_ref[r], preferred_element_type=jnp.float32)
            Os.append(lax.dot_general(A, vh, (((1,), (1,)), ((), ())),
                                      preferred_element_type=jnp.float32))
        mx = Os[0]
        for r in range(1, _R):
            mx = jnp.maximum(mx, Os[r])
        es = [jnp.exp(o - mx) for o in Os]
        se = es[0]
        for r in range(1, _R):
            se = se + es[r]
        num = jnp.zeros_like(se)
        for r in range(1, _R):
            num = num + float(r) * es[r]
        mhat_ref[...] = num / se

        tc = tcode_s[pl.ds(j * _BLK, _BLK)].astype(jnp.float32)
        obs = tc > 0.5
        ot = jnp.zeros_like(mx)
        for r in range(_R):
            ot = jnp.where(tc == float(r + 1), Os[r], ot)
        lterm = jnp.where(obs, mx + jnp.log(se) - ot, 0.0)

        pbest = Os[0]
        pcls = jnp.zeros_like(mx)
        for r in range(1, _R):
            gt = Os[r] > pbest
            pbest = jnp.where(gt, Os[r], pbest)
            pcls = jnp.where(gt, float(r), pcls)
        corr = jnp.where(obs & (pcls == (tc - 1.0)), 1.0, 0.0)

        ls = jnp.sum(lterm)
        nb = jnp.sum(jnp.where(obs, 1.0, 0.0))
        cr = jnp.sum(corr)

        @pl.when(j == 0)
        def _():
            sacc[0] = ls
            sacc[1] = nb
            sacc[2] = cr

        @pl.when(j != 0)
        def _():
            sacc[0] = sacc[0] + ls
            sacc[1] = sacc[1] + nb
            sacc[2] = sacc[2] + cr

        @pl.when(j == _NB - 1)
        def _():
            nbm = jnp.maximum(sacc[1], 1.0)
            loss_ref[...] = jnp.broadcast_to(sacc[0] / nbm, (1, 1))
            acc_ref[...] = jnp.broadcast_to(sacc[2] / nbm, (1, 1))


def _main_call(gcode, uemb, vemb, gcl_W, v_row, dense_W, db_row, gcl_b,
               bilin_W):
    return pl.pallas_call(
        _main_body,
        grid=(2 * _NB,),
        in_specs=[
            pl.BlockSpec((_BLK, _KP), lambda i: (jnp.minimum(i, _NB - 1), 0)),
            pl.BlockSpec((_BLK, _D), lambda i: (jnp.minimum(i, _NB - 1), 0)),
            pl.BlockSpec((_NV, _D), lambda i: (0, 0)),
            pl.BlockSpec((_R, _D, _H0), lambda i: (0, 0, 0)),
            pl.BlockSpec((1, _BV), lambda i: (0, 0)),
            pl.BlockSpec((_H0, _H1), lambda i: (0, 0)),
            pl.BlockSpec((1, _H1), lambda i: (0, 0)),
            pl.BlockSpec((_R, _H0), lambda i: (0, 0)),
            pl.BlockSpec((_R, _H1, _H1), lambda i: (0, 0, 0)),
        ],
        out_specs=[
            pl.BlockSpec((_BLK, _BV), lambda i: (jnp.maximum(i - _NB, 0), 0)),
            pl.BlockSpec((1, 1), lambda i: (0, 0)),
            pl.BlockSpec((1, 1), lambda i: (0, 0)),
        ],
        out_shape=[
            jax.ShapeDtypeStruct((_BU, _BV), jnp.float32),
            jax.ShapeDtypeStruct((1, 1), jnp.float32),
            jax.ShapeDtypeStruct((1, 1), jnp.float32),
        ],
        scratch_shapes=[
            pltpu.VMEM((_R, _KP, _H0), jnp.float32),
            pltpu.VMEM((_KP, 1), jnp.float32),
            pltpu.VMEM((_BU, _H0), jnp.float32),
            pltpu.VMEM((_BU, 1), jnp.float32),
            pltpu.VMEM((1, _KP), jnp.float32),
            pltpu.VMEM((_KP, _H0), jnp.float32),
            pltpu.VMEM((_BU, _BV), jnp.bfloat16),
            pltpu.VMEM((_BU, _H1), jnp.float32),
            pltpu.VMEM((_NV, _H1), jnp.float32),
            pltpu.SMEM((3,), jnp.float32),
        ],
        compiler_params=pltpu.CompilerParams(
            dimension_semantics=("arbitrary",)),
    )(gcode, uemb, vemb, gcl_W, v_row, dense_W, db_row, gcl_b, bilin_W)


def kernel(u, v, u_table, v_table, gcl_W, gcl_b, dense_W, dense_b, bilin_W,
           ratings):
    u = u.astype(jnp.int32)
    v = v.astype(jnp.int32)
    u2 = u.reshape(_NW, _GR_W)
    v2 = jnp.concatenate([v, jnp.zeros((_BVP - _BV,), jnp.int32)]
                         ).reshape(_NW, _VE_W)

    code = _compress_call(ratings)
    gcode, uemb, vemb_p = _sc_gather(code, u2, v2, u_table, v_table)
    vemb = vemb_p[:_NV]
    v_row = v.reshape(1, _BV)

    mhat, loss, acc = _main_call(gcode, uemb, vemb, gcl_W, v_row, dense_W,
                                 dense_b.reshape(1, _H1), gcl_b, bilin_W)
    return mhat, loss[0, 0], acc[0, 0]
